# baseline (speedup 1.0000x reference)
import functools

import jax
import jax.numpy as jnp
from jax import lax
from jax.experimental import pallas as pl
from jax.experimental.pallas import tpu as pltpu
from jax.experimental.pallas import tpu_sc as plsc

B, SEQ, PRED, E, K = 4096, 512, 96, 64, 8
A = B * K
T = 512
NT_A = B // T
TR = 2048
NT = A // TR
NS = NT + E
EPAD = 72
NEG = -1e30

NW = 32
TPW = B // NW


def _shift_cumsum(x, n, axis):
    total = x
    sh = 1
    while sh < n:
        if axis == 0:
            z = jnp.zeros((sh,) + x.shape[1:], x.dtype)
            total = total + jnp.concatenate([z, total[:-sh]], axis=0)
        else:
            z = jnp.zeros(x.shape[:1] + (sh,), x.dtype)
            total = total + jnp.concatenate([z, total[:, :-sh]], axis=1)
        sh *= 2
    return total - x


def _route_body(x_ref, wg_ref, bg_ref, ut_ref, w_ref, eid_ref,
                rank_ref, off_lo_ref, off_hi_ref, estep_ref, tstep_ref,
                loss_ref, cnt_acc, dacc, pacc):
    i = pl.program_id(0)
    x = x_ref[...]
    gate = jnp.dot(x, wg_ref[...], preferred_element_type=jnp.float32)
    gate = gate + bg_ref[...]

    gate_t = gate.T
    cur = gate_t
    vals = []
    onehots = []
    for _ in range(K):
        m = jnp.max(cur, axis=0, keepdims=True)
        oh = cur == m
        vals.append(m)
        onehots.append(oh)
        cur = jnp.where(oh, NEG, cur)
    v = jnp.concatenate(vals, axis=0)
    ev = jnp.exp(v - v[0:1, :])
    w_ref[...] = ev / jnp.sum(ev, axis=0, keepdims=True)

    ohs = jnp.zeros((E, T), jnp.float32)
    for k in range(K):
        ohs = ohs + onehots[k].astype(jnp.float32)
    excl = jnp.dot(ohs, ut_ref[...], preferred_element_type=jnp.float32)
    c_rank = jnp.where(i == 0, excl, cnt_acc[...] + excl)
    eio = lax.broadcasted_iota(jnp.int32, (E, T), 0).astype(jnp.float32)
    ranks = []
    eids = []
    for k in range(K):
        ohf = onehots[k].astype(jnp.float32)
        ranks.append(jnp.sum(ohf * c_rank, axis=0, keepdims=True))
        eids.append(jnp.sum(ohf * eio, axis=0, keepdims=True))
    rank_ref[...] = jnp.concatenate(ranks, axis=0).astype(jnp.int32)
    eid_ref[...] = jnp.concatenate(eids, axis=0).astype(jnp.int32)

    tile_cnt = jnp.sum(ohs, axis=1, keepdims=True)
    new_cnt = jnp.where(i == 0, tile_cnt, cnt_acc[...] + tile_cnt)
    cnt_acc[...] = new_cnt

    ex = jnp.exp(gate_t - vals[0])
    gp = ex / jnp.sum(ex, axis=0, keepdims=True)
    p_part = jnp.sum(gp, axis=1, keepdims=True)
    d_part = jnp.sum(onehots[0].astype(jnp.float32), axis=1, keepdims=True)
    dacc[...] = jnp.where(i == 0, d_part, dacc[...] + d_part)
    pacc[...] = jnp.where(i == 0, p_part, pacc[...] + p_part)

    @pl.when(i == pl.num_programs(0) - 1)
    def _fin():
        loss_ref[...] = jnp.sum(dacc[...] * pacc[...]).reshape(1, 1) * (E / (B * B))
        cnt_row = new_cnt.T
        off_lo = _shift_cumsum(cnt_row, E, axis=1)
        off_hi = off_lo + cnt_row
        zpad = jnp.zeros((1, EPAD - E), jnp.float32)
        off_lo_ref[...] = jnp.concatenate([off_lo, zpad], axis=1).astype(jnp.int32)
        off_hi_ref[...] = jnp.concatenate([off_hi, zpad], axis=1).astype(jnp.int32)

        r_lo = lax.broadcasted_iota(jnp.int32, (NT, 1), 0).astype(jnp.float32) * TR
        elo_f = jnp.sum((jnp.broadcast_to(off_hi, (NT, E)) <= r_lo)
                        .astype(jnp.float32), axis=1, keepdims=True)
        ehi_f = jnp.sum((jnp.broadcast_to(off_lo, (NT, E)) < r_lo + TR)
                        .astype(jnp.float32), axis=1, keepdims=True)
        estep_ref[...] = elo_f.astype(jnp.int32)
        tstep_ref[...] = ehi_f.astype(jnp.int32)


def _route(x, Wg, bg2, ut):
    return pl.pallas_call(
        _route_body,
        grid=(NT_A,),
        in_specs=[
            pl.BlockSpec((T, SEQ), lambda i: (i, 0)),
            pl.BlockSpec((SEQ, E), lambda i: (0, 0)),
            pl.BlockSpec((1, E), lambda i: (0, 0)),
            pl.BlockSpec((T, T), lambda i: (0, 0)),
        ],
        out_specs=[
            pl.BlockSpec((K, T), lambda i: (0, i)),
            pl.BlockSpec((K, T), lambda i: (0, i)),
            pl.BlockSpec((K, T), lambda i: (0, i)),
            pl.BlockSpec((1, EPAD), lambda i: (0, 0)),
            pl.BlockSpec((1, EPAD), lambda i: (0, 0)),
            pl.BlockSpec((NT, 1), lambda i: (0, 0)),
            pl.BlockSpec((NT, 1), lambda i: (0, 0)),
            pl.BlockSpec((1, 1), lambda i: (0, 0)),
        ],
        out_shape=[
            jax.ShapeDtypeStruct((K, B), jnp.float32),
            jax.ShapeDtypeStruct((K, B), jnp.int32),
            jax.ShapeDtypeStruct((K, B), jnp.int32),
            jax.ShapeDtypeStruct((1, EPAD), jnp.int32),
            jax.ShapeDtypeStruct((1, EPAD), jnp.int32),
            jax.ShapeDtypeStruct((NT, 1), jnp.int32),
            jax.ShapeDtypeStruct((NT, 1), jnp.int32),
            jax.ShapeDtypeStruct((1, 1), jnp.float32),
        ],
        scratch_shapes=[
            pltpu.VMEM((E, 1), jnp.float32),
            pltpu.VMEM((E, 1), jnp.float32),
            pltpu.VMEM((E, 1), jnp.float32),
        ],
    )(x, Wg, bg2, ut)


def _sc_mesh():
    return plsc.VectorSubcoreMesh(core_axis_name="c", subcore_axis_name="s")


def _dispatch_sc(x, eidx, rank, w, off_lo):
    @functools.partial(
        pl.kernel,
        out_type=[
            jax.ShapeDtypeStruct((A, SEQ + 128), jnp.float32),
            jax.ShapeDtypeStruct((K, B), jnp.int32),
        ],
        mesh=_sc_mesh(),
        scratch_types=[
            pltpu.VMEM((K, TPW), jnp.int32),
            pltpu.VMEM((K, TPW), jnp.int32),
            pltpu.VMEM((K, TPW), jnp.float32),
            pltpu.VMEM((K, TPW), jnp.int32),
            pltpu.VMEM((128,), jnp.int32),
            pltpu.VMEM((TPW, SEQ + 128), jnp.float32),
            pltpu.SemaphoreType.DMA,
        ],
        compiler_params=pltpu.CompilerParams(needs_layout_passes=False),
    )
    def body(x_h, eidx_h, rank_h, w_h, off_h, xs_h, pos_h,
             ev, rv, wv, posv, offv, myrows, sem_x):
        wid = lax.axis_index("s") * 2 + lax.axis_index("c")
        base = wid * TPW
        pltpu.sync_copy(off_h.at[pl.ds(0, E)], offv.at[pl.ds(0, E)])
        for k in range(K):
            pltpu.sync_copy(eidx_h.at[k, pl.ds(base, TPW)], ev.at[k])
            pltpu.sync_copy(rank_h.at[k, pl.ds(base, TPW)], rv.at[k])
            pltpu.sync_copy(w_h.at[k, pl.ds(base, TPW)], wv.at[k])
        pltpu.sync_copy(x_h.at[pl.ds(base, TPW)], myrows.at[:, pl.ds(0, SEQ)])
        lane = lax.iota(jnp.int32, 16)
        for k in range(K):
            for c in range(TPW // 16):
                sl = pl.ds(c * 16, 16)
                e16 = ev[k, sl]
                pos16 = plsc.load_gather(offv, [e16]) + rv[k, sl]
                posv[k, sl] = pos16
        for k in range(K):
            pltpu.sync_copy(posv.at[k], pos_h.at[k, pl.ds(base, TPW)])
        for k in range(K):
            for c in range(TPW // 16):
                sl = pl.ds(c * 16, 16)
                plsc.store_scatter(
                    myrows, [c * 16 + lane, jnp.full_like(lane, SEQ)],
                    wv[k, sl])
            pltpu.async_copy(myrows, xs_h.at[posv.at[k]], sem_x).wait()

    return body(x, eidx, rank, w, off_lo)


def _expert_body(elo, ehi, olo, ohi, xs_ref, we_ref, be_ref, ys_ref):
    t = pl.program_id(0)
    rows = t * TR + lax.broadcasted_iota(jnp.int32, (TR, 1), 0)
    xall = xs_ref[:, :SEQ].astype(jnp.bfloat16)
    wcol = xs_ref[:, SEQ:SEQ + 1]
    ys_ref[...] = jnp.zeros((TR, 128), jnp.float32)

    def per_expert(e, _):
        lo = olo[e]
        hi = ohi[e]
        mask = (rows >= lo) & (rows < hi)
        xm = jnp.where(mask, xall, jnp.bfloat16(0))
        wvec = jnp.where(mask, wcol, 0.0)
        y = jnp.dot(xm, we_ref[e].astype(jnp.bfloat16),
                    preferred_element_type=jnp.float32)
        contrib = wvec * (y + be_ref[e])
        contrib = jnp.concatenate(
            [contrib, jnp.zeros((TR, 128 - PRED), jnp.float32)], axis=1)
        ys_ref[...] += contrib
        return 0

    lax.fori_loop(elo[t], ehi[t], per_expert, 0)


def _expert_mm(elo, ehi, olo, ohi, xs, We, be):
    grid_spec = pltpu.PrefetchScalarGridSpec(
        num_scalar_prefetch=4,
        grid=(NT,),
        in_specs=[
            pl.BlockSpec((TR, SEQ + 128), lambda t, el, eh, lo, hi: (t, 0)),
            pl.BlockSpec((E, SEQ, PRED), lambda t, el, eh, lo, hi: (0, 0, 0)),
            pl.BlockSpec((E, PRED), lambda t, el, eh, lo, hi: (0, 0)),
        ],
        out_specs=pl.BlockSpec((TR, 128), lambda t, el, eh, lo, hi: (t, 0)),
    )
    return pl.pallas_call(
        _expert_body,
        grid_spec=grid_spec,
        out_shape=jax.ShapeDtypeStruct((A, 128), jnp.float32),
    )(elo, ehi, olo, ohi, xs, We, be)


def _combine_sc(ys, pos):
    @functools.partial(
        pl.kernel,
        out_type=jax.ShapeDtypeStruct((B, 128), jnp.float32),
        mesh=_sc_mesh(),
        scratch_types=[
            pltpu.VMEM((K, TPW), jnp.int32),
            pltpu.VMEM((TPW, 128), jnp.float32),
            pltpu.VMEM((TPW, 128), jnp.float32),
            pltpu.SemaphoreType.DMA,
        ],
        compiler_params=pltpu.CompilerParams(needs_layout_passes=False),
    )
    def body(ys_h, pos_h, out_h, posv, rows, acc, sem):
        wid = lax.axis_index("s") * 2 + lax.axis_index("c")
        base = wid * TPW
        for k in range(K):
            pltpu.sync_copy(pos_h.at[k, pl.ds(base, TPW)], posv.at[k])
        pltpu.async_copy(ys_h.at[posv.at[0]], acc, sem).wait()

        for k in range(1, K):
            pltpu.async_copy(ys_h.at[posv.at[k]], rows, sem).wait()

            def add_row(t, _):
                for c in range(128 // 16):
                    sl = pl.ds(c * 16, 16)
                    plsc.addupdate(acc.at[t, sl], rows[t, sl])
                return 0

            lax.fori_loop(0, TPW, add_row, 0)
        pltpu.sync_copy(acc, out_h.at[pl.ds(base, TPW)])

    return body(ys, pos)


@jax.jit
def _moe(x, Wg, bg2, We, be):
    tt = lax.broadcasted_iota(jnp.int32, (T, T), 0)
    ut = (tt < tt.T).astype(jnp.float32)
    (w, eidx, rank, off_lo, off_hi, estep, tstep, loss) = _route(
        x, Wg, bg2, ut)
    xs, pos = _dispatch_sc(x, eidx, rank, w, off_lo.reshape(EPAD))
    ys = _expert_mm(estep.reshape(NT), tstep.reshape(NT),
                    off_lo.reshape(EPAD), off_hi.reshape(EPAD),
                    xs, We, be)
    out = _combine_sc(ys, pos)
    return out[:, :PRED], loss[0, 0]


def kernel(x, Wg, bg, We, be):
    return _moe(x, Wg, bg.reshape(1, E), We, be)


# speedup vs baseline: 1.1258x; 1.1258x over previous
import functools

import jax
import jax.numpy as jnp
from jax import lax
from jax.experimental import pallas as pl
from jax.experimental.pallas import tpu as pltpu
from jax.experimental.pallas import tpu_sc as plsc

B, SEQ, PRED, E, K = 4096, 512, 96, 64, 8
A = B * K
T = 512
NT_A = B // T
TR = 1024
NT = A // TR
NS = NT + E
EPAD = 72
NEG = -1e30

NW = 32
TPW = B // NW


def _shift_cumsum(x, n, axis):
    total = x
    sh = 1
    while sh < n:
        if axis == 0:
            z = jnp.zeros((sh,) + x.shape[1:], x.dtype)
            total = total + jnp.concatenate([z, total[:-sh]], axis=0)
        else:
            z = jnp.zeros(x.shape[:1] + (sh,), x.dtype)
            total = total + jnp.concatenate([z, total[:, :-sh]], axis=1)
        sh *= 2
    return total - x


def _route_body(x_ref, wg_ref, bg_ref, ut_ref, w_ref, eid_ref,
                rank_ref, off_lo_ref, off_hi_ref, estep_ref, tstep_ref,
                loss_ref, cnt_acc, dacc, pacc):
    i = pl.program_id(0)
    x = x_ref[...]
    gate = jnp.dot(x, wg_ref[...], preferred_element_type=jnp.float32)
    gate = gate + bg_ref[...]

    gate_t = gate.T
    cur = gate_t
    vals = []
    onehots = []
    for _ in range(K):
        m = jnp.max(cur, axis=0, keepdims=True)
        oh = cur == m
        vals.append(m)
        onehots.append(oh)
        cur = jnp.where(oh, NEG, cur)
    v = jnp.concatenate(vals, axis=0)
    ev = jnp.exp(v - v[0:1, :])
    w_ref[...] = ev / jnp.sum(ev, axis=0, keepdims=True)

    ohs = jnp.zeros((E, T), jnp.float32)
    for k in range(K):
        ohs = ohs + onehots[k].astype(jnp.float32)
    excl = jnp.dot(ohs, ut_ref[...], preferred_element_type=jnp.float32)
    c_rank = jnp.where(i == 0, excl, cnt_acc[...] + excl)
    eio = lax.broadcasted_iota(jnp.int32, (E, T), 0).astype(jnp.float32)
    ranks = []
    eids = []
    for k in range(K):
        ohf = onehots[k].astype(jnp.float32)
        ranks.append(jnp.sum(ohf * c_rank, axis=0, keepdims=True))
        eids.append(jnp.sum(ohf * eio, axis=0, keepdims=True))
    rank_ref[...] = jnp.concatenate(ranks, axis=0).astype(jnp.int32)
    eid_ref[...] = jnp.concatenate(eids, axis=0).astype(jnp.int32)

    tile_cnt = jnp.sum(ohs, axis=1, keepdims=True)
    new_cnt = jnp.where(i == 0, tile_cnt, cnt_acc[...] + tile_cnt)
    cnt_acc[...] = new_cnt

    ex = jnp.exp(gate_t - vals[0])
    gp = ex / jnp.sum(ex, axis=0, keepdims=True)
    p_part = jnp.sum(gp, axis=1, keepdims=True)
    d_part = jnp.sum(onehots[0].astype(jnp.float32), axis=1, keepdims=True)
    dacc[...] = jnp.where(i == 0, d_part, dacc[...] + d_part)
    pacc[...] = jnp.where(i == 0, p_part, pacc[...] + p_part)

    @pl.when(i == pl.num_programs(0) - 1)
    def _fin():
        loss_ref[...] = jnp.sum(dacc[...] * pacc[...]).reshape(1, 1) * (E / (B * B))
        cnt_row = new_cnt.T
        off_lo = _shift_cumsum(cnt_row, E, axis=1)
        off_hi = off_lo + cnt_row
        zpad = jnp.zeros((1, EPAD - E), jnp.float32)
        off_lo_ref[...] = jnp.concatenate([off_lo, zpad], axis=1).astype(jnp.int32)
        off_hi_ref[...] = jnp.concatenate([off_hi, zpad], axis=1).astype(jnp.int32)

        r_lo = lax.broadcasted_iota(jnp.int32, (NT, 1), 0).astype(jnp.float32) * TR
        elo_f = jnp.sum((jnp.broadcast_to(off_hi, (NT, E)) <= r_lo)
                        .astype(jnp.float32), axis=1, keepdims=True)
        ehi_f = jnp.sum((jnp.broadcast_to(off_lo, (NT, E)) < r_lo + TR)
                        .astype(jnp.float32), axis=1, keepdims=True)
        estep_ref[...] = elo_f.astype(jnp.int32)
        tstep_ref[...] = ehi_f.astype(jnp.int32)


def _route(x, Wg, bg2, ut):
    return pl.pallas_call(
        _route_body,
        grid=(NT_A,),
        in_specs=[
            pl.BlockSpec((T, SEQ), lambda i: (i, 0)),
            pl.BlockSpec((SEQ, E), lambda i: (0, 0)),
            pl.BlockSpec((1, E), lambda i: (0, 0)),
            pl.BlockSpec((T, T), lambda i: (0, 0)),
        ],
        out_specs=[
            pl.BlockSpec((K, T), lambda i: (0, i)),
            pl.BlockSpec((K, T), lambda i: (0, i)),
            pl.BlockSpec((K, T), lambda i: (0, i)),
            pl.BlockSpec((1, EPAD), lambda i: (0, 0)),
            pl.BlockSpec((1, EPAD), lambda i: (0, 0)),
            pl.BlockSpec((NT, 1), lambda i: (0, 0)),
            pl.BlockSpec((NT, 1), lambda i: (0, 0)),
            pl.BlockSpec((1, 1), lambda i: (0, 0)),
        ],
        out_shape=[
            jax.ShapeDtypeStruct((K, B), jnp.float32),
            jax.ShapeDtypeStruct((K, B), jnp.int32),
            jax.ShapeDtypeStruct((K, B), jnp.int32),
            jax.ShapeDtypeStruct((1, EPAD), jnp.int32),
            jax.ShapeDtypeStruct((1, EPAD), jnp.int32),
            jax.ShapeDtypeStruct((NT, 1), jnp.int32),
            jax.ShapeDtypeStruct((NT, 1), jnp.int32),
            jax.ShapeDtypeStruct((1, 1), jnp.float32),
        ],
        scratch_shapes=[
            pltpu.VMEM((E, 1), jnp.float32),
            pltpu.VMEM((E, 1), jnp.float32),
            pltpu.VMEM((E, 1), jnp.float32),
        ],
    )(x, Wg, bg2, ut)


def _sc_mesh():
    return plsc.VectorSubcoreMesh(core_axis_name="c", subcore_axis_name="s")


def _dispatch_sc(x, eidx, rank, w, off_lo):
    @functools.partial(
        pl.kernel,
        out_type=[
            jax.ShapeDtypeStruct((A, SEQ + 128), jnp.float32),
            jax.ShapeDtypeStruct((K, B), jnp.int32),
        ],
        mesh=_sc_mesh(),
        scratch_types=[
            pltpu.VMEM((K, TPW), jnp.int32),
            pltpu.VMEM((K, TPW), jnp.int32),
            pltpu.VMEM((K, TPW), jnp.float32),
            pltpu.VMEM((K, TPW), jnp.int32),
            pltpu.VMEM((128,), jnp.int32),
            pltpu.VMEM((TPW, SEQ + 128), jnp.float32),
            pltpu.SemaphoreType.DMA,
        ],
        compiler_params=pltpu.CompilerParams(needs_layout_passes=False),
    )
    def body(x_h, eidx_h, rank_h, w_h, off_h, xs_h, pos_h,
             ev, rv, wv, posv, offv, myrows, sem_x):
        wid = lax.axis_index("s") * 2 + lax.axis_index("c")
        base = wid * TPW
        pltpu.sync_copy(off_h.at[pl.ds(0, E)], offv.at[pl.ds(0, E)])
        for k in range(K):
            pltpu.sync_copy(eidx_h.at[k, pl.ds(base, TPW)], ev.at[k])
            pltpu.sync_copy(rank_h.at[k, pl.ds(base, TPW)], rv.at[k])
            pltpu.sync_copy(w_h.at[k, pl.ds(base, TPW)], wv.at[k])
        pltpu.sync_copy(x_h.at[pl.ds(base, TPW)], myrows.at[:, pl.ds(0, SEQ)])
        lane = lax.iota(jnp.int32, 16)
        for k in range(K):
            for c in range(TPW // 16):
                sl = pl.ds(c * 16, 16)
                e16 = ev[k, sl]
                pos16 = plsc.load_gather(offv, [e16]) + rv[k, sl]
                posv[k, sl] = pos16
        for k in range(K):
            pltpu.sync_copy(posv.at[k], pos_h.at[k, pl.ds(base, TPW)])
        for k in range(K):
            for c in range(TPW // 16):
                sl = pl.ds(c * 16, 16)
                plsc.store_scatter(
                    myrows, [c * 16 + lane, jnp.full_like(lane, SEQ)],
                    wv[k, sl])
            pltpu.async_copy(myrows, xs_h.at[posv.at[k]], sem_x).wait()

    return body(x, eidx, rank, w, off_lo)


def _expert_body(elo, ehi, olo, ohi, xs_ref, we_ref, be_ref, ys_ref):
    t = pl.program_id(0)
    rows = t * TR + lax.broadcasted_iota(jnp.int32, (TR, 1), 0)
    xall = xs_ref[:, :SEQ].astype(jnp.bfloat16)
    wcol = xs_ref[:, SEQ:SEQ + 1]
    ys_ref[...] = jnp.zeros((TR, 128), jnp.float32)

    def per_expert(e, _):
        lo = olo[e]
        hi = ohi[e]
        mask = (rows >= lo) & (rows < hi)
        xm = jnp.where(mask, xall, jnp.bfloat16(0))
        wvec = jnp.where(mask, wcol, 0.0)
        y = jnp.dot(xm, we_ref[e].astype(jnp.bfloat16),
                    preferred_element_type=jnp.float32)
        contrib = wvec * (y + be_ref[e])
        contrib = jnp.concatenate(
            [contrib, jnp.zeros((TR, 128 - PRED), jnp.float32)], axis=1)
        ys_ref[...] += contrib
        return 0

    lax.fori_loop(elo[t], ehi[t], per_expert, 0)


def _expert_mm(elo, ehi, olo, ohi, xs, We, be):
    grid_spec = pltpu.PrefetchScalarGridSpec(
        num_scalar_prefetch=4,
        grid=(NT,),
        in_specs=[
            pl.BlockSpec((TR, SEQ + 128), lambda t, el, eh, lo, hi: (t, 0)),
            pl.BlockSpec((E, SEQ, PRED), lambda t, el, eh, lo, hi: (0, 0, 0)),
            pl.BlockSpec((E, PRED), lambda t, el, eh, lo, hi: (0, 0)),
        ],
        out_specs=pl.BlockSpec((TR, 128), lambda t, el, eh, lo, hi: (t, 0)),
    )
    return pl.pallas_call(
        _expert_body,
        grid_spec=grid_spec,
        out_shape=jax.ShapeDtypeStruct((A, 128), jnp.float32),
    )(elo, ehi, olo, ohi, xs, We, be)


def _combine_sc(ys, pos):
    @functools.partial(
        pl.kernel,
        out_type=jax.ShapeDtypeStruct((B, 128), jnp.float32),
        mesh=_sc_mesh(),
        scratch_types=[
            pltpu.VMEM((K, TPW), jnp.int32),
            pltpu.VMEM((TPW, 128), jnp.float32),
            pltpu.VMEM((TPW, 128), jnp.float32),
            pltpu.VMEM((TPW, 128), jnp.float32),
            pltpu.SemaphoreType.DMA,
            pltpu.SemaphoreType.DMA,
        ],
        compiler_params=pltpu.CompilerParams(needs_layout_passes=False),
    )
    def body(ys_h, pos_h, out_h, posv, rows0, rows1, acc, sem0, sem1):
        wid = lax.axis_index("s") * 2 + lax.axis_index("c")
        base = wid * TPW
        for k in range(K):
            pltpu.sync_copy(pos_h.at[k, pl.ds(base, TPW)], posv.at[k])
        bufs = [rows0, rows1]
        sems = [sem0, sem1]
        cp_acc = pltpu.async_copy(ys_h.at[posv.at[0]], acc, sems[0])
        cps = [None, None]
        cps[1] = pltpu.async_copy(ys_h.at[posv.at[1]], bufs[1], sems[1])
        cp_acc.wait()
        for k in range(1, K):
            cur = k % 2
            cps[cur].wait()
            if k + 1 < K:
                nxt = (k + 1) % 2
                cps[nxt] = pltpu.async_copy(
                    ys_h.at[posv.at[k + 1]], bufs[nxt], sems[nxt])
            src = bufs[cur]

            def add_row(t, _, src=src):
                for c in range(128 // 16):
                    sl = pl.ds(c * 16, 16)
                    plsc.addupdate(acc.at[t, sl], src[t, sl])
                return 0

            lax.fori_loop(0, TPW, add_row, 0)
        pltpu.sync_copy(acc, out_h.at[pl.ds(base, TPW)])

    return body(ys, pos)


@jax.jit
def _moe(x, Wg, bg2, We, be):
    tt = lax.broadcasted_iota(jnp.int32, (T, T), 0)
    ut = (tt < tt.T).astype(jnp.float32)
    (w, eidx, rank, off_lo, off_hi, estep, tstep, loss) = _route(
        x, Wg, bg2, ut)
    xs, pos = _dispatch_sc(x, eidx, rank, w, off_lo.reshape(EPAD))
    ys = _expert_mm(estep.reshape(NT), tstep.reshape(NT),
                    off_lo.reshape(EPAD), off_hi.reshape(EPAD),
                    xs, We, be)
    out = _combine_sc(ys, pos)
    return out[:, :PRED], loss[0, 0]


def kernel(x, Wg, bg, We, be):
    return _moe(x, Wg, bg.reshape(1, E), We, be)
